# Initial kernel scaffold; baseline (speedup 1.0000x reference)
import jax
import jax.numpy as jnp
from jax.experimental import pallas as pl


def kernel(x, edge_index, lmax, batch, W1, b1, W2, b2, W3, b3, Wfc, bfc):
    raise NotImplementedError("write your pallas kernel here")


# speedup vs baseline: 9.7931x; 9.7931x over previous
import functools

import jax
import jax.numpy as jnp
from jax import lax
from jax.experimental import pallas as pl
from jax.experimental.pallas import tpu as pltpu
from jax.experimental.pallas import tpu_sc as plsc

N = 10000
E = 320000
DUMMY = N
N_ACC = 10112
NTILE = 32
CHUNK = 128
E_PAD = 327680
KCH = E_PAD // (NTILE * CHUNK)
STRIPE = N_ACC // 16
BLK = 2000
GRID = N // BLK
NBUF = 4


@functools.cache
def _sc_scatter_fn(width):
    mesh = plsc.VectorSubcoreMesh(core_axis_name="c", subcore_axis_name="s")

    @functools.partial(
        pl.kernel,
        mesh=mesh,
        out_type=jax.ShapeDtypeStruct((2, N_ACC, width), jnp.float32),
        compiler_params=pltpu.CompilerParams(use_tc_tiling_on_sc=False),
        scratch_types=[
            pltpu.VMEM((KCH, CHUNK), jnp.int32),
            pltpu.VMEM((KCH, CHUNK), jnp.int32),
        ] + [pltpu.VMEM((CHUNK, width), jnp.float32)] * NBUF + [
            pltpu.VMEM_SHARED((N_ACC, width), jnp.float32),
        ] + [pltpu.SemaphoreType.DMA] * NBUF,
    )
    def sc_kernel(table_hbm, gidx_hbm, sidx_hbm, zeros_hbm, out_hbm,
                  gidx_v, sidx_v, *rest):
        rows_bufs = rest[:NBUF]
        acc = rest[NBUF]
        sems = rest[NBUF + 1:]
        cid = lax.axis_index("c")
        sid = lax.axis_index("s")
        wid = cid * 16 + sid
        pltpu.sync_copy(zeros_hbm, acc.at[pl.ds(sid * STRIPE, STRIPE)])
        pltpu.sync_copy(gidx_hbm.at[wid], gidx_v)
        pltpu.sync_copy(sidx_hbm.at[wid], sidx_v)
        plsc.subcore_barrier()

        for b in range(NBUF):
            pltpu.async_copy(table_hbm.at[gidx_v.at[b]], rows_bufs[b], sems[b])

        def body(jb, carry):
            j = NBUF * jb
            for off in range(NBUF):
                rows, sem = rows_bufs[off], sems[off]
                pltpu.make_async_copy(
                    table_hbm.at[gidx_v.at[j + off]], rows, sem).wait()
                pltpu.sync_copy(rows, acc.at[sidx_v.at[j + off]], add=True)

                @pl.when(j + off + NBUF < KCH)
                def _():
                    pltpu.async_copy(
                        table_hbm.at[gidx_v.at[j + off + NBUF]], rows, sem)
            return carry

        lax.fori_loop(0, KCH // NBUF, body, 0)
        plsc.subcore_barrier()
        pltpu.sync_copy(acc.at[pl.ds(sid * STRIPE, STRIPE)],
                        out_hbm.at[cid].at[pl.ds(sid * STRIPE, STRIPE)])

    return sc_kernel


def _sc_scatter(table, gidx, sidx, zeros):
    return _sc_scatter_fn(table.shape[1])(table, gidx, sidx, zeros)


def _row_spec(w):
    return pl.BlockSpec((BLK, w), lambda i: (i, 0))


def _s_spec(w):
    return pl.BlockSpec((2, BLK, w), lambda i: (0, i, 0))


def _full_spec(shape):
    nd = len(shape)
    return pl.BlockSpec(shape, lambda i, _n=nd: (0,) * _n)


def _prep_edges(src_p, dst_p):
    def body(s_ref, d_ref, dsc_ref, gsc_ref):
        s = s_ref[...]
        d = d_ref[...]
        loop = s == d
        dsc_ref[...] = jnp.where(loop, DUMMY, d)
        gsc_ref[...] = jnp.where(loop, DUMMY, s)

    rows = E_PAD // 128
    spec = pl.BlockSpec((rows // 5, 128), lambda i: (i, 0))
    return pl.pallas_call(
        body,
        grid=(5,),
        in_specs=[spec, spec],
        out_specs=[spec, spec],
        out_shape=[jax.ShapeDtypeStruct((rows, 128), jnp.int32)] * 2,
    )(src_p, dst_p)


def _dinv_kernel(deg_out):
    def body(deg_ref, dinv_ref):
        d = deg_ref[0, :, 0:8] + deg_ref[1, :, 0:8]
        dinv_ref[...] = jnp.where(
            d > 0, lax.rsqrt(jnp.maximum(d, 1.0)), 0.0)

    return pl.pallas_call(
        body,
        grid=(GRID,),
        in_specs=[_s_spec(16)],
        out_specs=_row_spec(8),
        out_shape=jax.ShapeDtypeStruct((N, 8), jnp.float32),
    )(deg_out)


def _l1_coeffs(x, W1, dinv):
    def body(x_ref, w_ref, dinv_ref, c_ref, u_ref):
        xb = x_ref[...]
        dv = dinv_ref[:, 0:1]
        for k in range(4):
            ck = jnp.dot(xb, w_ref[k], preferred_element_type=jnp.float32)
            c_ref[:, k * 32:(k + 1) * 32] = ck
            if k == 3:
                u_ref[...] = dv * ck

    return pl.pallas_call(
        body,
        grid=(GRID,),
        in_specs=[_row_spec(128), _full_spec(W1.shape), _row_spec(8)],
        out_specs=[_row_spec(128), _row_spec(32)],
        out_shape=[jax.ShapeDtypeStruct((N, 128), jnp.float32),
                   jax.ShapeDtypeStruct((N, 32), jnp.float32)],
    )(x, W1, dinv)


def _l1_bstep(C, s, prev, dinv, coef, *, ci, sub_prev2):
    def body(c_ref, s_ref, q_ref, dinv_ref, coef_ref, b_ref, u_ref):
        a = coef_ref[0, 0]
        diag = coef_ref[0, 1]
        dv = dinv_ref[:, 0:1]
        S = s_ref[0] + s_ref[1]
        b = c_ref[:, ci * 32:(ci + 1) * 32] + 2.0 * (a * dv * S + diag * q_ref[...])
        if sub_prev2:
            b = b - c_ref[:, 96:128]
        b_ref[...] = b
        u_ref[...] = dv * b

    return pl.pallas_call(
        body,
        grid=(GRID,),
        in_specs=[_row_spec(128), _s_spec(32), _row_spec(32), _row_spec(8),
                  _full_spec((1, 2))],
        out_specs=[_row_spec(32), _row_spec(32)],
        out_shape=[jax.ShapeDtypeStruct((N, 32), jnp.float32)] * 2,
    )(C, s, prev, dinv, coef)


def _l1_out(C, s, b1v, b2v, dinv, coef, bias1, W2):
    def body(c_ref, s_ref, b1_ref, b2_ref, dinv_ref, coef_ref, bias_ref,
             w_ref, h_ref, acc_ref, u_ref):
        a = coef_ref[0, 0]
        diag = coef_ref[0, 1]
        dv = dinv_ref[:, 0:1]
        S = s_ref[0] + s_ref[1]
        h = jnp.tanh(c_ref[:, 0:32] + a * dv * S + diag * b1_ref[...]
                     - b2_ref[...] + bias_ref[0:1, :])
        h_ref[...] = h
        acc_ref[...] = jnp.dot(h, w_ref[0], preferred_element_type=jnp.float32)
        u_ref[...] = dv * h

    return pl.pallas_call(
        body,
        grid=(GRID,),
        in_specs=[_row_spec(128), _s_spec(32), _row_spec(32), _row_spec(32),
                  _row_spec(8), _full_spec((1, 2)), _full_spec((1, 32)),
                  _full_spec(W2.shape)],
        out_specs=[_row_spec(32), _row_spec(64), _row_spec(32)],
        out_shape=[jax.ShapeDtypeStruct((N, 32), jnp.float32),
                   jax.ShapeDtypeStruct((N, 64), jnp.float32),
                   jax.ShapeDtypeStruct((N, 32), jnp.float32)],
    )(C, s, b1v, b2v, dinv, coef, bias1, W2)


def _fwd_step(T1, T0, s, acc, Wfull, dinv, coef, *, k, fin, first):
    def body(t1_ref, t0_ref, s_ref, acc_ref, w_ref, dinv_ref, coef_ref,
             tn_ref, accn_ref, u_ref):
        a = coef_ref[0, 0]
        diag = coef_ref[0, 1]
        dv = dinv_ref[:, 0:1]
        S = s_ref[0] + s_ref[1]
        t = a * dv * S + diag * t1_ref[...]
        if not first:
            t = 2.0 * t - t0_ref[...]
        tn_ref[...] = t
        accn_ref[...] = acc_ref[...] + jnp.dot(
            t, w_ref[k], preferred_element_type=jnp.float32)
        u_ref[...] = dv * t

    return pl.pallas_call(
        body,
        grid=(GRID,),
        in_specs=[_row_spec(fin), _row_spec(fin), _s_spec(fin), _row_spec(64),
                  _full_spec(Wfull.shape), _row_spec(8), _full_spec((1, 2))],
        out_specs=[_row_spec(fin), _row_spec(64), _row_spec(fin)],
        out_shape=[jax.ShapeDtypeStruct((N, fin), jnp.float32),
                   jax.ShapeDtypeStruct((N, 64), jnp.float32),
                   jax.ShapeDtypeStruct((N, fin), jnp.float32)],
    )(T1, T0, s, acc, Wfull, dinv, coef)


def _layer_close(T2, T1, s, acc, Wfull, dinv, coef, bias, Wnext, *, fin):
    def body(t2_ref, t1_ref, s_ref, acc_ref, w_ref, dinv_ref, coef_ref,
             bias_ref, wn_ref, h_ref, accn_ref, u_ref):
        a = coef_ref[0, 0]
        diag = coef_ref[0, 1]
        dv = dinv_ref[:, 0:1]
        S = s_ref[0] + s_ref[1]
        t3 = 2.0 * (a * dv * S + diag * t2_ref[...]) - t1_ref[...]
        h = jnp.tanh(acc_ref[...] + jnp.dot(
            t3, w_ref[3], preferred_element_type=jnp.float32) + bias_ref[0:1, :])
        h_ref[...] = h
        accn_ref[...] = jnp.dot(h, wn_ref[0], preferred_element_type=jnp.float32)
        u_ref[...] = dv * h

    return pl.pallas_call(
        body,
        grid=(GRID,),
        in_specs=[_row_spec(fin), _row_spec(fin), _s_spec(fin), _row_spec(64),
                  _full_spec(Wfull.shape), _row_spec(8), _full_spec((1, 2)),
                  _full_spec((1, 64)), _full_spec(Wnext.shape)],
        out_specs=[_row_spec(64), _row_spec(64), _row_spec(64)],
        out_shape=[jax.ShapeDtypeStruct((N, 64), jnp.float32)] * 3,
    )(T2, T1, s, acc, Wfull, dinv, coef, bias, Wnext)


def _final(T2, T1, s, acc, W3, dinv, coef, bias3, Wfc, bfc):
    def body(t2_ref, t1_ref, s_ref, acc_ref, w_ref, dinv_ref, coef_ref,
             bias_ref, wfc_ref, bfc_ref, out_ref, pool_ref):
        i = pl.program_id(0)
        a = coef_ref[0, 0]
        diag = coef_ref[0, 1]
        dv = dinv_ref[:, 0:1]
        S = s_ref[0] + s_ref[1]
        t3 = 2.0 * (a * dv * S + diag * t2_ref[...]) - t1_ref[...]
        h = jnp.tanh(acc_ref[...] + jnp.dot(
            t3, w_ref[3], preferred_element_type=jnp.float32) + bias_ref[0:1, :])
        part = jnp.sum(h, axis=0, keepdims=True)

        @pl.when(i == 0)
        def _():
            pool_ref[...] = jnp.zeros_like(pool_ref)

        pool_ref[0:1, :] += part

        @pl.when(i == GRID - 1)
        def _():
            out_ref[...] = jnp.tanh(
                jnp.dot(pool_ref[0:1, :], wfc_ref[...],
                        preferred_element_type=jnp.float32) + bfc_ref[...])

    return pl.pallas_call(
        body,
        grid=(GRID,),
        in_specs=[_row_spec(64), _row_spec(64), _s_spec(64), _row_spec(64),
                  _full_spec(W3.shape), _row_spec(8), _full_spec((1, 2)),
                  _full_spec((1, 64)), _full_spec(Wfc.shape),
                  _full_spec((1, 10))],
        out_specs=pl.BlockSpec((1, 10), lambda i: (0, 0)),
        out_shape=jax.ShapeDtypeStruct((1, 10), jnp.float32),
        scratch_shapes=[pltpu.VMEM((1, 64), jnp.float32)],
    )(T2, T1, s, acc, W3, dinv, coef, bias3, Wfc, bfc)


def kernel(x, edge_index, lmax, batch, W1, b1, W2, b2, W3, b3, Wfc, bfc):
    src, dst = edge_index[0], edge_index[1]
    pad = E_PAD - E
    src_p = jnp.concatenate([src, jnp.zeros((pad,), jnp.int32)])
    dst_p = jnp.concatenate([dst, jnp.zeros((pad,), jnp.int32)])
    rows = E_PAD // 128
    dst_sc, deg_sc = _prep_edges(src_p.reshape(rows, 128),
                                 dst_p.reshape(rows, 128))

    gidx = src_p.reshape(NTILE, KCH, CHUNK)
    sidx_spmv = dst_sc.reshape(NTILE, KCH, CHUNK)
    sidx_deg = deg_sc.reshape(NTILE, KCH, CHUNK)

    z16 = jnp.zeros((STRIPE, 16), jnp.float32)
    z32 = jnp.zeros((STRIPE, 32), jnp.float32)
    z64 = jnp.zeros((STRIPE, 64), jnp.float32)

    ones_tab = jnp.ones((N, 16), jnp.float32)
    deg_out = _sc_scatter(ones_tab, gidx, sidx_deg, z16)
    dinv = _dinv_kernel(deg_out)

    lam = lmax[0]
    a = -2.0 / lam
    diag = 2.0 / lam - 1.0
    coef = jnp.stack([a, diag]).reshape(1, 2)

    C, u3 = _l1_coeffs(x, W1, dinv)
    s3 = _sc_scatter(u3, gidx, sidx_spmv, z32)
    b2v, u2 = _l1_bstep(C, s3, C[:, 96:128], dinv, coef, ci=2, sub_prev2=False)
    s2 = _sc_scatter(u2, gidx, sidx_spmv, z32)
    b1v, u1 = _l1_bstep(C, s2, b2v, dinv, coef, ci=1, sub_prev2=True)
    s1 = _sc_scatter(u1, gidx, sidx_spmv, z32)
    h1, acc2, u = _l1_out(C, s1, b1v, b2v, dinv, coef, b1.reshape(1, 32), W2)

    r1 = _sc_scatter(u, gidx, sidx_spmv, z32)
    T1, acc2, u = _fwd_step(h1, h1, r1, acc2, W2, dinv, coef,
                            k=1, fin=32, first=True)
    r2 = _sc_scatter(u, gidx, sidx_spmv, z32)
    T2, acc2, u = _fwd_step(T1, h1, r2, acc2, W2, dinv, coef,
                            k=2, fin=32, first=False)
    r3 = _sc_scatter(u, gidx, sidx_spmv, z32)
    h2, acc3, u = _layer_close(T2, T1, r3, acc2, W2, dinv, coef,
                               b2.reshape(1, 64), W3, fin=32)

    q1 = _sc_scatter(u, gidx, sidx_spmv, z64)
    T1, acc3, u = _fwd_step(h2, h2, q1, acc3, W3, dinv, coef,
                            k=1, fin=64, first=True)
    q2 = _sc_scatter(u, gidx, sidx_spmv, z64)
    T2, acc3, u = _fwd_step(T1, h2, q2, acc3, W3, dinv, coef,
                            k=2, fin=64, first=False)
    q3 = _sc_scatter(u, gidx, sidx_spmv, z64)

    return _final(T2, T1, q3, acc3, W3, dinv, coef, b3.reshape(1, 64),
                  Wfc, bfc.reshape(1, 10))


# speedup vs baseline: 9.8991x; 1.0108x over previous
import functools

import jax
import jax.numpy as jnp
from jax import lax
from jax.experimental import pallas as pl
from jax.experimental.pallas import tpu as pltpu
from jax.experimental.pallas import tpu_sc as plsc

N = 10000
E = 320000
DUMMY = N
N_ACC = 10112
NTILE = 32
CHUNK = 128
E_PAD = 327680
KCH = E_PAD // (NTILE * CHUNK)
STRIPE = N_ACC // 16
BLK = 2000
GRID = N // BLK
NBUF = 8
DEPTH = 4


@functools.cache
def _sc_scatter_fn(width):
    mesh = plsc.VectorSubcoreMesh(core_axis_name="c", subcore_axis_name="s")

    @functools.partial(
        pl.kernel,
        mesh=mesh,
        out_type=jax.ShapeDtypeStruct((2, N_ACC, width), jnp.float32),
        compiler_params=pltpu.CompilerParams(use_tc_tiling_on_sc=False),
        scratch_types=[
            pltpu.VMEM((KCH, CHUNK), jnp.int32),
            pltpu.VMEM((KCH, CHUNK), jnp.int32),
        ] + [pltpu.VMEM((CHUNK, width), jnp.float32)] * NBUF
          + [pltpu.VMEM_SHARED((N_ACC, width), jnp.float32)]
          + [pltpu.SemaphoreType.DMA] * (2 * NBUF + 1),
    )
    def sc_kernel(table_hbm, gidx_hbm, sidx_hbm, zeros_hbm, out_hbm,
                  gidx_v, sidx_v, *rest):
        rows_bufs = rest[:NBUF]
        acc = rest[NBUF]
        gsems = rest[NBUF + 1:2 * NBUF + 1]
        ssems = rest[2 * NBUF + 1:3 * NBUF + 1]
        psem = rest[3 * NBUF + 1]
        cid = lax.axis_index("c")
        sid = lax.axis_index("s")
        wid = cid * 16 + sid
        cp_z = pltpu.async_copy(zeros_hbm, acc.at[pl.ds(sid * STRIPE, STRIPE)],
                                psem)
        cp_g = pltpu.async_copy(gidx_hbm.at[wid], gidx_v, psem)
        cp_s = pltpu.async_copy(sidx_hbm.at[wid], sidx_v, psem)
        cp_z.wait()
        cp_g.wait()
        cp_s.wait()
        plsc.subcore_barrier()

        def _g(t, b):
            return pltpu.make_async_copy(
                table_hbm.at[gidx_v.at[t]], rows_bufs[b], gsems[b])

        def _s(t, b):
            return pltpu.make_async_copy(
                rows_bufs[b], acc.at[sidx_v.at[t]], ssems[b])

        for t in range(DEPTH):
            _g(t, t).start()

        def body(lap, carry):
            j0 = lap * NBUF
            for off in range(NBUF):
                jj = j0 + off
                _g(jj, off).wait()
                _s(jj, off).start()
                bb = (off + DEPTH) % NBUF

                @pl.when(jj + DEPTH < KCH)
                def _():
                    @pl.when(jj + DEPTH >= NBUF)
                    def _():
                        _s(0, bb).wait()
                    _g(jj + DEPTH, bb).start()
            return carry

        lax.fori_loop(0, KCH // NBUF, body, 0)
        for b in range(NBUF):
            _s(0, b).wait()
        plsc.subcore_barrier()
        pltpu.sync_copy(acc.at[pl.ds(sid * STRIPE, STRIPE)],
                        out_hbm.at[cid].at[pl.ds(sid * STRIPE, STRIPE)])

    return sc_kernel


def _sc_scatter(table, gidx, sidx, zeros):
    return _sc_scatter_fn(table.shape[1])(table, gidx, sidx, zeros)


def _row_spec(w):
    return pl.BlockSpec((BLK, w), lambda i: (i, 0))


def _s_spec(w):
    return pl.BlockSpec((2, BLK, w), lambda i: (0, i, 0))


def _full_spec(shape):
    nd = len(shape)
    return pl.BlockSpec(shape, lambda i, _n=nd: (0,) * _n)


def _prep_edges(src_p, dst_p):
    def body(s_ref, d_ref, dsc_ref, gsc_ref):
        s = s_ref[...]
        d = d_ref[...]
        loop = s == d
        dsc_ref[...] = jnp.where(loop, DUMMY, d)
        gsc_ref[...] = jnp.where(loop, DUMMY, s)

    rows = E_PAD // 128
    spec = pl.BlockSpec((rows // 5, 128), lambda i: (i, 0))
    return pl.pallas_call(
        body,
        grid=(5,),
        in_specs=[spec, spec],
        out_specs=[spec, spec],
        out_shape=[jax.ShapeDtypeStruct((rows, 128), jnp.int32)] * 2,
    )(src_p, dst_p)


def _dinv_kernel(deg_out):
    def body(deg_ref, dinv_ref):
        d = deg_ref[0, :, 0:8] + deg_ref[1, :, 0:8]
        dinv_ref[...] = jnp.where(
            d > 0, lax.rsqrt(jnp.maximum(d, 1.0)), 0.0)

    return pl.pallas_call(
        body,
        grid=(GRID,),
        in_specs=[_s_spec(16)],
        out_specs=_row_spec(8),
        out_shape=jax.ShapeDtypeStruct((N, 8), jnp.float32),
    )(deg_out)


def _l1_coeffs(x, W1, dinv):
    def body(x_ref, w_ref, dinv_ref, c_ref, u_ref):
        xb = x_ref[...]
        dv = dinv_ref[:, 0:1]
        for k in range(4):
            ck = jnp.dot(xb, w_ref[k], preferred_element_type=jnp.float32)
            c_ref[:, k * 32:(k + 1) * 32] = ck
            if k == 3:
                u_ref[...] = dv * ck

    return pl.pallas_call(
        body,
        grid=(GRID,),
        in_specs=[_row_spec(128), _full_spec(W1.shape), _row_spec(8)],
        out_specs=[_row_spec(128), _row_spec(32)],
        out_shape=[jax.ShapeDtypeStruct((N, 128), jnp.float32),
                   jax.ShapeDtypeStruct((N, 32), jnp.float32)],
    )(x, W1, dinv)


def _l1_bstep(C, s, prev, dinv, coef, *, ci, sub_prev2):
    def body(c_ref, s_ref, q_ref, dinv_ref, coef_ref, b_ref, u_ref):
        a = coef_ref[0, 0]
        diag = coef_ref[0, 1]
        dv = dinv_ref[:, 0:1]
        S = s_ref[0] + s_ref[1]
        b = c_ref[:, ci * 32:(ci + 1) * 32] + 2.0 * (a * dv * S + diag * q_ref[...])
        if sub_prev2:
            b = b - c_ref[:, 96:128]
        b_ref[...] = b
        u_ref[...] = dv * b

    return pl.pallas_call(
        body,
        grid=(GRID,),
        in_specs=[_row_spec(128), _s_spec(32), _row_spec(32), _row_spec(8),
                  _full_spec((1, 2))],
        out_specs=[_row_spec(32), _row_spec(32)],
        out_shape=[jax.ShapeDtypeStruct((N, 32), jnp.float32)] * 2,
    )(C, s, prev, dinv, coef)


def _l1_out(C, s, b1v, b2v, dinv, coef, bias1, W2):
    def body(c_ref, s_ref, b1_ref, b2_ref, dinv_ref, coef_ref, bias_ref,
             w_ref, h_ref, acc_ref, u_ref):
        a = coef_ref[0, 0]
        diag = coef_ref[0, 1]
        dv = dinv_ref[:, 0:1]
        S = s_ref[0] + s_ref[1]
        h = jnp.tanh(c_ref[:, 0:32] + a * dv * S + diag * b1_ref[...]
                     - b2_ref[...] + bias_ref[0:1, :])
        h_ref[...] = h
        acc_ref[...] = jnp.dot(h, w_ref[0], preferred_element_type=jnp.float32)
        u_ref[...] = dv * h

    return pl.pallas_call(
        body,
        grid=(GRID,),
        in_specs=[_row_spec(128), _s_spec(32), _row_spec(32), _row_spec(32),
                  _row_spec(8), _full_spec((1, 2)), _full_spec((1, 32)),
                  _full_spec(W2.shape)],
        out_specs=[_row_spec(32), _row_spec(64), _row_spec(32)],
        out_shape=[jax.ShapeDtypeStruct((N, 32), jnp.float32),
                   jax.ShapeDtypeStruct((N, 64), jnp.float32),
                   jax.ShapeDtypeStruct((N, 32), jnp.float32)],
    )(C, s, b1v, b2v, dinv, coef, bias1, W2)


def _fwd_step(T1, T0, s, acc, Wfull, dinv, coef, *, k, fin, first):
    def body(t1_ref, t0_ref, s_ref, acc_ref, w_ref, dinv_ref, coef_ref,
             tn_ref, accn_ref, u_ref):
        a = coef_ref[0, 0]
        diag = coef_ref[0, 1]
        dv = dinv_ref[:, 0:1]
        S = s_ref[0] + s_ref[1]
        t = a * dv * S + diag * t1_ref[...]
        if not first:
            t = 2.0 * t - t0_ref[...]
        tn_ref[...] = t
        accn_ref[...] = acc_ref[...] + jnp.dot(
            t, w_ref[k], preferred_element_type=jnp.float32)
        u_ref[...] = dv * t

    return pl.pallas_call(
        body,
        grid=(GRID,),
        in_specs=[_row_spec(fin), _row_spec(fin), _s_spec(fin), _row_spec(64),
                  _full_spec(Wfull.shape), _row_spec(8), _full_spec((1, 2))],
        out_specs=[_row_spec(fin), _row_spec(64), _row_spec(fin)],
        out_shape=[jax.ShapeDtypeStruct((N, fin), jnp.float32),
                   jax.ShapeDtypeStruct((N, 64), jnp.float32),
                   jax.ShapeDtypeStruct((N, fin), jnp.float32)],
    )(T1, T0, s, acc, Wfull, dinv, coef)


def _layer_close(T2, T1, s, acc, Wfull, dinv, coef, bias, Wnext, *, fin):
    def body(t2_ref, t1_ref, s_ref, acc_ref, w_ref, dinv_ref, coef_ref,
             bias_ref, wn_ref, h_ref, accn_ref, u_ref):
        a = coef_ref[0, 0]
        diag = coef_ref[0, 1]
        dv = dinv_ref[:, 0:1]
        S = s_ref[0] + s_ref[1]
        t3 = 2.0 * (a * dv * S + diag * t2_ref[...]) - t1_ref[...]
        h = jnp.tanh(acc_ref[...] + jnp.dot(
            t3, w_ref[3], preferred_element_type=jnp.float32) + bias_ref[0:1, :])
        h_ref[...] = h
        accn_ref[...] = jnp.dot(h, wn_ref[0], preferred_element_type=jnp.float32)
        u_ref[...] = dv * h

    return pl.pallas_call(
        body,
        grid=(GRID,),
        in_specs=[_row_spec(fin), _row_spec(fin), _s_spec(fin), _row_spec(64),
                  _full_spec(Wfull.shape), _row_spec(8), _full_spec((1, 2)),
                  _full_spec((1, 64)), _full_spec(Wnext.shape)],
        out_specs=[_row_spec(64), _row_spec(64), _row_spec(64)],
        out_shape=[jax.ShapeDtypeStruct((N, 64), jnp.float32)] * 3,
    )(T2, T1, s, acc, Wfull, dinv, coef, bias, Wnext)


def _final(T2, T1, s, acc, W3, dinv, coef, bias3, Wfc, bfc):
    def body(t2_ref, t1_ref, s_ref, acc_ref, w_ref, dinv_ref, coef_ref,
             bias_ref, wfc_ref, bfc_ref, out_ref, pool_ref):
        i = pl.program_id(0)
        a = coef_ref[0, 0]
        diag = coef_ref[0, 1]
        dv = dinv_ref[:, 0:1]
        S = s_ref[0] + s_ref[1]
        t3 = 2.0 * (a * dv * S + diag * t2_ref[...]) - t1_ref[...]
        h = jnp.tanh(acc_ref[...] + jnp.dot(
            t3, w_ref[3], preferred_element_type=jnp.float32) + bias_ref[0:1, :])
        part = jnp.sum(h, axis=0, keepdims=True)

        @pl.when(i == 0)
        def _():
            pool_ref[...] = jnp.zeros_like(pool_ref)

        pool_ref[0:1, :] += part

        @pl.when(i == GRID - 1)
        def _():
            out_ref[...] = jnp.tanh(
                jnp.dot(pool_ref[0:1, :], wfc_ref[...],
                        preferred_element_type=jnp.float32) + bfc_ref[...])

    return pl.pallas_call(
        body,
        grid=(GRID,),
        in_specs=[_row_spec(64), _row_spec(64), _s_spec(64), _row_spec(64),
                  _full_spec(W3.shape), _row_spec(8), _full_spec((1, 2)),
                  _full_spec((1, 64)), _full_spec(Wfc.shape),
                  _full_spec((1, 10))],
        out_specs=pl.BlockSpec((1, 10), lambda i: (0, 0)),
        out_shape=jax.ShapeDtypeStruct((1, 10), jnp.float32),
        scratch_shapes=[pltpu.VMEM((1, 64), jnp.float32)],
    )(T2, T1, s, acc, W3, dinv, coef, bias3, Wfc, bfc)


def kernel(x, edge_index, lmax, batch, W1, b1, W2, b2, W3, b3, Wfc, bfc):
    src, dst = edge_index[0], edge_index[1]
    pad = E_PAD - E
    src_p = jnp.concatenate([src, jnp.zeros((pad,), jnp.int32)])
    dst_p = jnp.concatenate([dst, jnp.zeros((pad,), jnp.int32)])
    rows = E_PAD // 128
    dst_sc, deg_sc = _prep_edges(src_p.reshape(rows, 128),
                                 dst_p.reshape(rows, 128))

    gidx = src_p.reshape(NTILE, KCH, CHUNK)
    sidx_spmv = dst_sc.reshape(NTILE, KCH, CHUNK)
    sidx_deg = deg_sc.reshape(NTILE, KCH, CHUNK)

    z16 = jnp.zeros((STRIPE, 16), jnp.float32)
    z32 = jnp.zeros((STRIPE, 32), jnp.float32)
    z64 = jnp.zeros((STRIPE, 64), jnp.float32)

    ones_tab = jnp.ones((N, 16), jnp.float32)
    deg_out = _sc_scatter(ones_tab, gidx, sidx_deg, z16)
    dinv = _dinv_kernel(deg_out)

    lam = lmax[0]
    a = -2.0 / lam
    diag = 2.0 / lam - 1.0
    coef = jnp.stack([a, diag]).reshape(1, 2)

    C, u3 = _l1_coeffs(x, W1, dinv)
    s3 = _sc_scatter(u3, gidx, sidx_spmv, z32)
    b2v, u2 = _l1_bstep(C, s3, C[:, 96:128], dinv, coef, ci=2, sub_prev2=False)
    s2 = _sc_scatter(u2, gidx, sidx_spmv, z32)
    b1v, u1 = _l1_bstep(C, s2, b2v, dinv, coef, ci=1, sub_prev2=True)
    s1 = _sc_scatter(u1, gidx, sidx_spmv, z32)
    h1, acc2, u = _l1_out(C, s1, b1v, b2v, dinv, coef, b1.reshape(1, 32), W2)

    r1 = _sc_scatter(u, gidx, sidx_spmv, z32)
    T1, acc2, u = _fwd_step(h1, h1, r1, acc2, W2, dinv, coef,
                            k=1, fin=32, first=True)
    r2 = _sc_scatter(u, gidx, sidx_spmv, z32)
    T2, acc2, u = _fwd_step(T1, h1, r2, acc2, W2, dinv, coef,
                            k=2, fin=32, first=False)
    r3 = _sc_scatter(u, gidx, sidx_spmv, z32)
    h2, acc3, u = _layer_close(T2, T1, r3, acc2, W2, dinv, coef,
                               b2.reshape(1, 64), W3, fin=32)

    q1 = _sc_scatter(u, gidx, sidx_spmv, z64)
    T1, acc3, u = _fwd_step(h2, h2, q1, acc3, W3, dinv, coef,
                            k=1, fin=64, first=True)
    q2 = _sc_scatter(u, gidx, sidx_spmv, z64)
    T2, acc3, u = _fwd_step(T1, h2, q2, acc3, W3, dinv, coef,
                            k=2, fin=64, first=False)
    q3 = _sc_scatter(u, gidx, sidx_spmv, z64)

    return _final(T2, T1, q3, acc3, W3, dinv, coef, b3.reshape(1, 64),
                  Wfc, bfc.reshape(1, 10))


# speedup vs baseline: 15.7654x; 1.5926x over previous
import functools

import jax
import jax.numpy as jnp
from jax import lax
from jax.experimental import pallas as pl
from jax.experimental.pallas import tpu as pltpu
from jax.experimental.pallas import tpu_sc as plsc

N = 10000
E = 320000
DUMMY = N
N_ACC = 10112
NTILE = 32
CHUNK = 128
E_PAD = 327680
KCH = E_PAD // (NTILE * CHUNK)
STRIPE = N_ACC // 16
BLK = 2000
GRID = N // BLK
NBUF = 4


@functools.cache
def _sc_scatter_fn(width, dtype):
    mesh = plsc.VectorSubcoreMesh(core_axis_name="c", subcore_axis_name="s")

    @functools.partial(
        pl.kernel,
        mesh=mesh,
        out_type=jax.ShapeDtypeStruct((2, N_ACC, width), dtype),
        compiler_params=pltpu.CompilerParams(use_tc_tiling_on_sc=False),
        scratch_types=[
            pltpu.VMEM((KCH, CHUNK), jnp.int32),
            pltpu.VMEM((KCH, CHUNK), jnp.int32),
        ] + [pltpu.VMEM((CHUNK, width), dtype)] * NBUF
          + [pltpu.VMEM_SHARED((N_ACC, width), dtype)]
          + [pltpu.SemaphoreType.DMA] * NBUF,
    )
    def sc_kernel(table_hbm, gidx_hbm, sidx_hbm, zeros_hbm, out_hbm,
                  gidx_v, sidx_v, *rest):
        rows_bufs = rest[:NBUF]
        acc = rest[NBUF]
        gsems = rest[NBUF + 1:2 * NBUF + 1]
        cid = lax.axis_index("c")
        sid = lax.axis_index("s")
        wid = cid * 16 + sid
        pltpu.sync_copy(zeros_hbm, acc.at[pl.ds(sid * STRIPE, STRIPE)])
        pltpu.sync_copy(gidx_hbm.at[wid], gidx_v)
        pltpu.sync_copy(sidx_hbm.at[wid], sidx_v)
        plsc.subcore_barrier()

        for b in range(NBUF):
            pltpu.async_copy(table_hbm.at[gidx_v.at[b]], rows_bufs[b],
                             gsems[b])

        def body(jb, carry):
            j = NBUF * jb
            for off in range(NBUF):
                rows, sem = rows_bufs[off], gsems[off]
                pltpu.make_async_copy(
                    table_hbm.at[gidx_v.at[j + off]], rows, sem).wait()
                pltpu.sync_copy(rows, acc.at[sidx_v.at[j + off]], add=True)

                @pl.when(j + off + NBUF < KCH)
                def _():
                    pltpu.async_copy(
                        table_hbm.at[gidx_v.at[j + off + NBUF]], rows, sem)
            return carry

        lax.fori_loop(0, KCH // NBUF, body, 0)
        plsc.subcore_barrier()
        pltpu.sync_copy(acc.at[pl.ds(sid * STRIPE, STRIPE)],
                        out_hbm.at[cid].at[pl.ds(sid * STRIPE, STRIPE)])

    return sc_kernel


def _sc_scatter(table, gidx, sidx, zeros):
    return _sc_scatter_fn(table.shape[1], table.dtype)(table, gidx, sidx, zeros)


def _row_spec(w):
    return pl.BlockSpec((BLK, w), lambda i: (i, 0))


def _s_spec(w):
    return pl.BlockSpec((2, BLK, w), lambda i: (0, i, 0))


def _full_spec(shape):
    nd = len(shape)
    return pl.BlockSpec(shape, lambda i, _n=nd: (0,) * _n)


def _prep_edges(src_p, dst_p):
    def body(s_ref, d_ref, dsc_ref, gsc_ref):
        s = s_ref[...]
        d = d_ref[...]
        loop = s == d
        dsc_ref[...] = jnp.where(loop, DUMMY, d)
        gsc_ref[...] = jnp.where(loop, DUMMY, s)

    rows = E_PAD // 128
    spec = pl.BlockSpec((rows // 5, 128), lambda i: (i, 0))
    return pl.pallas_call(
        body,
        grid=(5,),
        in_specs=[spec, spec],
        out_specs=[spec, spec],
        out_shape=[jax.ShapeDtypeStruct((rows, 128), jnp.int32)] * 2,
    )(src_p, dst_p)


def _dinv_kernel(deg_out):
    def body(deg_ref, dinv_ref):
        d = deg_ref[0, :, 0:8] + deg_ref[1, :, 0:8]
        dinv_ref[...] = jnp.where(
            d > 0, lax.rsqrt(jnp.maximum(d, 1.0)), 0.0)

    return pl.pallas_call(
        body,
        grid=(GRID,),
        in_specs=[_s_spec(16)],
        out_specs=_row_spec(8),
        out_shape=jax.ShapeDtypeStruct((N, 8), jnp.float32),
    )(deg_out)


def _l1_coeffs(x, W1, dinv):
    def body(x_ref, w_ref, dinv_ref, c_ref, u_ref):
        xb = x_ref[...]
        dv = dinv_ref[:, 0:1]
        for k in range(4):
            ck = jnp.dot(xb, w_ref[k], preferred_element_type=jnp.float32)
            c_ref[:, k * 32:(k + 1) * 32] = ck
            if k == 3:
                u_ref[...] = (dv * ck).astype(jnp.bfloat16)

    return pl.pallas_call(
        body,
        grid=(GRID,),
        in_specs=[_row_spec(128), _full_spec(W1.shape), _row_spec(8)],
        out_specs=[_row_spec(128), _row_spec(32)],
        out_shape=[jax.ShapeDtypeStruct((N, 128), jnp.float32),
                   jax.ShapeDtypeStruct((N, 32), jnp.bfloat16)],
    )(x, W1, dinv)


def _l1_bstep(C, s, prev, dinv, coef, *, ci, sub_prev2):
    def body(c_ref, s_ref, q_ref, dinv_ref, coef_ref, b_ref, u_ref):
        a = coef_ref[0, 0]
        diag = coef_ref[0, 1]
        dv = dinv_ref[:, 0:1]
        S = s_ref[0].astype(jnp.float32) + s_ref[1].astype(jnp.float32)
        b = c_ref[:, ci * 32:(ci + 1) * 32] + 2.0 * (a * dv * S + diag * q_ref[...])
        if sub_prev2:
            b = b - c_ref[:, 96:128]
        b_ref[...] = b
        u_ref[...] = (dv * b).astype(jnp.bfloat16)

    return pl.pallas_call(
        body,
        grid=(GRID,),
        in_specs=[_row_spec(128), _s_spec(32), _row_spec(32), _row_spec(8),
                  _full_spec((1, 2))],
        out_specs=[_row_spec(32), _row_spec(32)],
        out_shape=[jax.ShapeDtypeStruct((N, 32), jnp.float32),
                   jax.ShapeDtypeStruct((N, 32), jnp.bfloat16)],
    )(C, s, prev, dinv, coef)


def _l1_out(C, s, b1v, b2v, dinv, coef, bias1, W2):
    def body(c_ref, s_ref, b1_ref, b2_ref, dinv_ref, coef_ref, bias_ref,
             w_ref, h_ref, acc_ref, u_ref):
        a = coef_ref[0, 0]
        diag = coef_ref[0, 1]
        dv = dinv_ref[:, 0:1]
        S = s_ref[0].astype(jnp.float32) + s_ref[1].astype(jnp.float32)
        h = jnp.tanh(c_ref[:, 0:32] + a * dv * S + diag * b1_ref[...]
                     - b2_ref[...] + bias_ref[0:1, :])
        h_ref[...] = h
        acc_ref[...] = jnp.dot(h, w_ref[0], preferred_element_type=jnp.float32)
        u_ref[...] = (dv * h).astype(jnp.bfloat16)

    return pl.pallas_call(
        body,
        grid=(GRID,),
        in_specs=[_row_spec(128), _s_spec(32), _row_spec(32), _row_spec(32),
                  _row_spec(8), _full_spec((1, 2)), _full_spec((1, 32)),
                  _full_spec(W2.shape)],
        out_specs=[_row_spec(32), _row_spec(64), _row_spec(32)],
        out_shape=[jax.ShapeDtypeStruct((N, 32), jnp.float32),
                   jax.ShapeDtypeStruct((N, 64), jnp.float32),
                   jax.ShapeDtypeStruct((N, 32), jnp.bfloat16)],
    )(C, s, b1v, b2v, dinv, coef, bias1, W2)


def _fwd_step(T1, T0, s, acc, Wfull, dinv, coef, *, k, fin, first):
    def body(t1_ref, t0_ref, s_ref, acc_ref, w_ref, dinv_ref, coef_ref,
             tn_ref, accn_ref, u_ref):
        a = coef_ref[0, 0]
        diag = coef_ref[0, 1]
        dv = dinv_ref[:, 0:1]
        S = s_ref[0].astype(jnp.float32) + s_ref[1].astype(jnp.float32)
        t = a * dv * S + diag * t1_ref[...]
        if not first:
            t = 2.0 * t - t0_ref[...]
        tn_ref[...] = t
        accn_ref[...] = acc_ref[...] + jnp.dot(
            t, w_ref[k], preferred_element_type=jnp.float32)
        u_ref[...] = (dv * t).astype(jnp.bfloat16)

    return pl.pallas_call(
        body,
        grid=(GRID,),
        in_specs=[_row_spec(fin), _row_spec(fin), _s_spec(fin), _row_spec(64),
                  _full_spec(Wfull.shape), _row_spec(8), _full_spec((1, 2))],
        out_specs=[_row_spec(fin), _row_spec(64), _row_spec(fin)],
        out_shape=[jax.ShapeDtypeStruct((N, fin), jnp.float32),
                   jax.ShapeDtypeStruct((N, 64), jnp.float32),
                   jax.ShapeDtypeStruct((N, fin), jnp.bfloat16)],
    )(T1, T0, s, acc, Wfull, dinv, coef)


def _layer_close(T2, T1, s, acc, Wfull, dinv, coef, bias, Wnext, *, fin):
    def body(t2_ref, t1_ref, s_ref, acc_ref, w_ref, dinv_ref, coef_ref,
             bias_ref, wn_ref, h_ref, accn_ref, u_ref):
        a = coef_ref[0, 0]
        diag = coef_ref[0, 1]
        dv = dinv_ref[:, 0:1]
        S = s_ref[0].astype(jnp.float32) + s_ref[1].astype(jnp.float32)
        t3 = 2.0 * (a * dv * S + diag * t2_ref[...]) - t1_ref[...]
        h = jnp.tanh(acc_ref[...] + jnp.dot(
            t3, w_ref[3], preferred_element_type=jnp.float32) + bias_ref[0:1, :])
        h_ref[...] = h
        accn_ref[...] = jnp.dot(h, wn_ref[0], preferred_element_type=jnp.float32)
        u_ref[...] = (dv * h).astype(jnp.bfloat16)

    return pl.pallas_call(
        body,
        grid=(GRID,),
        in_specs=[_row_spec(fin), _row_spec(fin), _s_spec(fin), _row_spec(64),
                  _full_spec(Wfull.shape), _row_spec(8), _full_spec((1, 2)),
                  _full_spec((1, 64)), _full_spec(Wnext.shape)],
        out_specs=[_row_spec(64), _row_spec(64), _row_spec(64)],
        out_shape=[jax.ShapeDtypeStruct((N, 64), jnp.float32),
                   jax.ShapeDtypeStruct((N, 64), jnp.float32),
                   jax.ShapeDtypeStruct((N, 64), jnp.bfloat16)],
    )(T2, T1, s, acc, Wfull, dinv, coef, bias, Wnext)


def _final(T2, T1, s, acc, W3, dinv, coef, bias3, Wfc, bfc):
    def body(t2_ref, t1_ref, s_ref, acc_ref, w_ref, dinv_ref, coef_ref,
             bias_ref, wfc_ref, bfc_ref, out_ref, pool_ref):
        i = pl.program_id(0)
        a = coef_ref[0, 0]
        diag = coef_ref[0, 1]
        dv = dinv_ref[:, 0:1]
        S = s_ref[0].astype(jnp.float32) + s_ref[1].astype(jnp.float32)
        t3 = 2.0 * (a * dv * S + diag * t2_ref[...]) - t1_ref[...]
        h = jnp.tanh(acc_ref[...] + jnp.dot(
            t3, w_ref[3], preferred_element_type=jnp.float32) + bias_ref[0:1, :])
        part = jnp.sum(h, axis=0, keepdims=True)

        @pl.when(i == 0)
        def _():
            pool_ref[...] = jnp.zeros_like(pool_ref)

        pool_ref[0:1, :] += part

        @pl.when(i == GRID - 1)
        def _():
            out_ref[...] = jnp.tanh(
                jnp.dot(pool_ref[0:1, :], wfc_ref[...],
                        preferred_element_type=jnp.float32) + bfc_ref[...])

    return pl.pallas_call(
        body,
        grid=(GRID,),
        in_specs=[_row_spec(64), _row_spec(64), _s_spec(64), _row_spec(64),
                  _full_spec(W3.shape), _row_spec(8), _full_spec((1, 2)),
                  _full_spec((1, 64)), _full_spec(Wfc.shape),
                  _full_spec((1, 10))],
        out_specs=pl.BlockSpec((1, 10), lambda i: (0, 0)),
        out_shape=jax.ShapeDtypeStruct((1, 10), jnp.float32),
        scratch_shapes=[pltpu.VMEM((1, 64), jnp.float32)],
    )(T2, T1, s, acc, W3, dinv, coef, bias3, Wfc, bfc)


def kernel(x, edge_index, lmax, batch, W1, b1, W2, b2, W3, b3, Wfc, bfc):
    src, dst = edge_index[0], edge_index[1]
    pad = E_PAD - E
    src_p = jnp.concatenate([src, jnp.zeros((pad,), jnp.int32)])
    dst_p = jnp.concatenate([dst, jnp.zeros((pad,), jnp.int32)])
    rows = E_PAD // 128
    dst_sc, deg_sc = _prep_edges(src_p.reshape(rows, 128),
                                 dst_p.reshape(rows, 128))

    gidx = src_p.reshape(NTILE, KCH, CHUNK)
    sidx_spmv = dst_sc.reshape(NTILE, KCH, CHUNK)
    sidx_deg = deg_sc.reshape(NTILE, KCH, CHUNK)

    z16 = jnp.zeros((STRIPE, 16), jnp.float32)
    z32 = jnp.zeros((STRIPE, 32), jnp.bfloat16)
    z64 = jnp.zeros((STRIPE, 64), jnp.bfloat16)

    ones_tab = jnp.ones((N, 16), jnp.float32)
    deg_out = _sc_scatter(ones_tab, gidx, sidx_deg, z16)
    dinv = _dinv_kernel(deg_out)

    lam = lmax[0]
    a = -2.0 / lam
    diag = 2.0 / lam - 1.0
    coef = jnp.stack([a, diag]).reshape(1, 2)

    C, u3 = _l1_coeffs(x, W1, dinv)
    s3 = _sc_scatter(u3, gidx, sidx_spmv, z32)
    b2v, u2 = _l1_bstep(C, s3, C[:, 96:128], dinv, coef, ci=2, sub_prev2=False)
    s2 = _sc_scatter(u2, gidx, sidx_spmv, z32)
    b1v, u1 = _l1_bstep(C, s2, b2v, dinv, coef, ci=1, sub_prev2=True)
    s1 = _sc_scatter(u1, gidx, sidx_spmv, z32)
    h1, acc2, u = _l1_out(C, s1, b1v, b2v, dinv, coef, b1.reshape(1, 32), W2)

    r1 = _sc_scatter(u, gidx, sidx_spmv, z32)
    T1, acc2, u = _fwd_step(h1, h1, r1, acc2, W2, dinv, coef,
                            k=1, fin=32, first=True)
    r2 = _sc_scatter(u, gidx, sidx_spmv, z32)
    T2, acc2, u = _fwd_step(T1, h1, r2, acc2, W2, dinv, coef,
                            k=2, fin=32, first=False)
    r3 = _sc_scatter(u, gidx, sidx_spmv, z32)
    h2, acc3, u = _layer_close(T2, T1, r3, acc2, W2, dinv, coef,
                               b2.reshape(1, 64), W3, fin=32)

    q1 = _sc_scatter(u, gidx, sidx_spmv, z64)
    T1, acc3, u = _fwd_step(h2, h2, q1, acc3, W3, dinv, coef,
                            k=1, fin=64, first=True)
    q2 = _sc_scatter(u, gidx, sidx_spmv, z64)
    T2, acc3, u = _fwd_step(T1, h2, q2, acc3, W3, dinv, coef,
                            k=2, fin=64, first=False)
    q3 = _sc_scatter(u, gidx, sidx_spmv, z64)

    return _final(T2, T1, q3, acc3, W3, dinv, coef, b3.reshape(1, 64),
                  Wfc, bfc.reshape(1, 10))
